# baseline (speedup 1.0000x reference)
import functools
import math

import jax
import jax.numpy as jnp
import numpy as np
from jax import lax
from jax.experimental import pallas as pl
from jax.experimental.pallas import tpu as pltpu
from jax.experimental.pallas import tpu_sc as plsc

N_NODES = 10000
N_EDGES = 10000
D1 = 32
D2 = 32
IN_VDIM = 128
IN_EDIM = 64
OUT_VDIM = 128
OUT_EDIM = 64
DHID = 64
HEADS = 4
NUM_INDS = 4
WDIM = 16
HD = DHID // HEADS
HP = 8
KV_PAD = 256

_MH = np.zeros((DHID, HP), np.float32)
for _h in range(HEADS):
    _MH[_h * HD:(_h + 1) * HD, _h] = 1.0 / math.sqrt(DHID)
_EH = np.zeros((HP, DHID), np.float32)
for _h in range(HEADS):
    _EH[_h, _h * HD:(_h + 1) * HD] = 1.0
_E0 = np.zeros((HEADS * NUM_INDS, NUM_INDS * DHID), np.float32)
for _i in range(NUM_INDS):
    for _h in range(HEADS):
        _E0[_h * NUM_INDS + _i, _i * DHID + _h * HD:_i * DHID + (_h + 1) * HD] = 1.0
_QC_MASK = np.zeros((DHID, HEADS * NUM_INDS), np.float32)
for _c in range(HEADS * NUM_INDS):
    _QC_MASK[(_c // NUM_INDS) * HD:(_c // NUM_INDS + 1) * HD, _c] = 1.0 / math.sqrt(DHID)
_QC_COLMAP = np.array([c % NUM_INDS for c in range(HEADS * NUM_INDS)], np.int32)


def _prep_consts(params):
    c = {}
    c["Mh"] = jnp.asarray(_MH)
    c["Eh"] = jnp.asarray(_EH)
    c["E0"] = jnp.asarray(_E0)
    I4 = jnp.asarray(np.eye(NUM_INDS, dtype=np.float32))
    c["MhBD"] = jnp.kron(I4, c["Mh"])
    c["EhBD"] = jnp.kron(I4, c["Eh"])
    c["J64"] = jnp.kron(
        jnp.ones((NUM_INDS, 1), jnp.float32),
        jnp.asarray(np.eye(DHID, dtype=np.float32)))
    for name in ("isab0", "isab1"):
        p = params[name]
        m0, m1 = p["mab0"], p["mab1"]
        c[name + "_AkvqT"] = jnp.concatenate(
            [m0["k"]["W"], m0["v"]["W"], m1["q"]["W"]], axis=0).T
        c[name + "_bkvq"] = jnp.concatenate(
            [m0["k"]["b"], m0["v"]["b"], m1["q"]["b"]])[None]
        Qp0 = p["I"][0] @ m0["q"]["W"].T + m0["q"]["b"]
        c[name + "_Qp0f"] = Qp0.reshape(1, NUM_INDS * DHID)
        c[name + "_Qc"] = Qp0.T[:, _QC_COLMAP] * _QC_MASK
        c[name + "_BDo0"] = jnp.kron(I4, m0["o"]["W"].T)
        c[name + "_bo0t"] = jnp.tile(m0["o"]["b"], NUM_INDS)[None]
        c[name + "_BDk1"] = jnp.kron(I4, m1["k"]["W"].T)
        c[name + "_bk1t"] = jnp.tile(m1["k"]["b"], NUM_INDS)[None]
        c[name + "_BDv1"] = jnp.kron(I4, m1["v"]["W"].T)
        c[name + "_bv1t"] = jnp.tile(m1["v"]["b"], NUM_INDS)[None]
        c[name + "_Wo1T"] = m1["o"]["W"].T
        c[name + "_bo1"] = m1["o"]["b"][None]
    WpeT = params["pe_v"]["W"].T
    bpe = params["pe_v"]["b"][None]
    c["Akvq_pe"] = WpeT @ c["isab0_AkvqT"]
    c["bkvq_pe"] = bpe @ c["isab0_AkvqT"] + c["isab0_bkvq"]
    pd = params["dec_mab"]
    c["WqdT"] = pd["q"]["W"].T
    c["bqd"] = pd["q"]["b"][None]
    c["AkvdT"] = jnp.concatenate([pd["k"]["W"], pd["v"]["W"]], axis=0).T
    c["bkvd"] = jnp.concatenate([pd["k"]["b"], pd["v"]["b"]])[None]
    c["WodT"] = pd["o"]["W"].T
    c["bod"] = pd["o"]["b"][None]
    c["WdlT"] = params["dec_lin"]["W"].T
    c["bdl"] = params["dec_lin"]["b"][None]
    c["WkvT"] = jnp.concatenate(
        [params["ke_lin"]["W"], params["ve_lin"]["W"],
         jnp.zeros((KV_PAD - DHID - OUT_VDIM, DHID), jnp.float32)], axis=0).T
    c["bkv"] = jnp.concatenate(
        [params["ke_lin"]["b"], params["ve_lin"]["b"],
         jnp.zeros((KV_PAD - DHID - OUT_VDIM,), jnp.float32)])[None]
    c["WqvT"] = params["qv_lin"]["W"].T
    c["bqv"] = params["qv_lin"]["b"][None]
    return c


def _sc_gather(table, idx, chunk):
    B = idx.shape[0]
    D = table.shape[1]
    NW = 32
    b_per_w = B // NW
    n_chunks = b_per_w // chunk
    n_pairs = n_chunks // 2
    mesh = plsc.VectorSubcoreMesh(core_axis_name="c", subcore_axis_name="s")

    @functools.partial(
        pl.kernel,
        out_type=jax.ShapeDtypeStruct((B, D), jnp.float32),
        mesh=mesh,
        scratch_types=[
            pltpu.VMEM((b_per_w,), jnp.int32),
            pltpu.VMEM((chunk, D), jnp.float32),
            pltpu.VMEM((chunk, D), jnp.float32),
            pltpu.SemaphoreType.DMA,
            pltpu.SemaphoreType.DMA,
            pltpu.SemaphoreType.DMA,
        ],
    )
    def gk(table_hbm, idx_hbm, out_hbm, idx_v, rows0, rows1,
           gsem, wsem0, wsem1):
        wid = lax.axis_index("s") * 2 + lax.axis_index("c")
        base0 = wid * b_per_w
        pltpu.sync_copy(idx_hbm.at[pl.ds(base0, b_per_w)], idx_v)

        def pair(p, carry):
            o0 = 2 * p * chunk
            o1 = o0 + chunk
            @pl.when(p > 0)
            def _():
                pltpu.make_async_copy(
                    rows0, out_hbm.at[pl.ds(base0 + o0, chunk)], wsem0).wait()
            pltpu.async_copy(
                table_hbm.at[idx_v.at[pl.ds(o0, chunk)]], rows0, gsem).wait()
            pltpu.async_copy(
                rows0, out_hbm.at[pl.ds(base0 + o0, chunk)], wsem0)
            @pl.when(p > 0)
            def _():
                pltpu.make_async_copy(
                    rows1, out_hbm.at[pl.ds(base0 + o1, chunk)], wsem1).wait()
            pltpu.async_copy(
                table_hbm.at[idx_v.at[pl.ds(o1, chunk)]], rows1, gsem).wait()
            pltpu.async_copy(
                rows1, out_hbm.at[pl.ds(base0 + o1, chunk)], wsem1)
            return carry

        lax.fori_loop(0, n_pairs, pair, 0)
        last = base0 + (n_chunks - 2) * chunk
        pltpu.make_async_copy(
            rows0, out_hbm.at[pl.ds(last, chunk)], wsem0).wait()
        pltpu.make_async_copy(
            rows1, out_hbm.at[pl.ds(last + chunk, chunk)], wsem1).wait()

    return gk(table, idx)


def _isab(P, Bb, cr, pref):
    M = D1
    BM = Bb * M
    W = NUM_INDS * DHID
    Kp0 = P[:, :DHID]
    Vp0 = P[:, DHID:2 * DHID]
    Qp1 = P[:, 2 * DHID:]
    S0 = (Kp0 @ cr[pref + "_Qc"]).reshape(Bb, M, HEADS * NUM_INDS)
    S0 = S0 - jnp.max(S0, axis=1, keepdims=True)
    A0 = jnp.exp(S0)
    A0 = A0 / jnp.sum(A0, axis=1, keepdims=True)
    AX = A0.reshape(BM, HEADS * NUM_INDS) @ cr["E0"]
    V4 = jnp.concatenate([Vp0] * NUM_INDS, axis=1)
    Hw = cr[pref + "_Qp0f"] + jnp.sum((AX * V4).reshape(Bb, M, W), axis=1)
    Hw = Hw + jnp.maximum(Hw @ cr[pref + "_BDo0"] + cr[pref + "_bo0t"], 0.0)
    Kall = Hw @ cr[pref + "_BDk1"] + cr[pref + "_bk1t"]
    Vall = Hw @ cr[pref + "_BDv1"] + cr[pref + "_bv1t"]
    Q4 = jnp.concatenate([Qp1] * NUM_INDS, axis=1)
    T = Q4.reshape(Bb, M, W) * Kall[:, None, :]
    S1 = (T.reshape(BM, W) @ cr["MhBD"]).reshape(Bb, M, NUM_INDS * HP)
    Sj = [S1[:, :, HP * j:HP * (j + 1)] for j in range(NUM_INDS)]
    mx = jnp.maximum(jnp.maximum(Sj[0], Sj[1]), jnp.maximum(Sj[2], Sj[3]))
    Ej = [jnp.exp(s - mx) for s in Sj]
    den = (Ej[0] + Ej[1]) + (Ej[2] + Ej[3])
    Aall = jnp.concatenate([e / den for e in Ej], axis=-1)
    Axf = Aall.reshape(BM, NUM_INDS * HP) @ cr["EhBD"]
    prod = Axf.reshape(Bb, M, W) * Vall[:, None, :]
    O1 = Qp1 + prod.reshape(BM, W) @ cr["J64"]
    return O1 + jnp.maximum(O1 @ cr[pref + "_Wo1T"] + cr[pref + "_bo1"], 0.0)


_S1_CONST_NAMES = (
    "Mh", "Eh", "E0", "MhBD", "EhBD", "J64",
    "isab0_AkvqT", "Akvq_pe", "bkvq_pe", "isab0_Qp0f", "isab0_Qc",
    "isab0_BDo0", "isab0_bo0t", "isab0_BDk1", "isab0_bk1t",
    "isab0_BDv1", "isab0_bv1t", "isab0_Wo1T", "isab0_bo1",
    "isab1_AkvqT", "isab1_bkvq", "isab1_Qp0f", "isab1_Qc",
    "isab1_BDo0", "isab1_bo0t", "isab1_BDk1", "isab1_bk1t",
    "isab1_BDv1", "isab1_bv1t", "isab1_Wo1T", "isab1_bo1",
    "WqdT", "bqd", "AkvdT", "bkvd", "WodT", "bod", "WdlT", "bdl",
    "WkvT", "bkv",
)


def _stage1_body(rows_ref, wgt_ref, ef_ref, *rest):
    const_refs = rest[:len(_S1_CONST_NAMES)]
    efn_ref, kv_ref = rest[len(_S1_CONST_NAMES):]
    cr = {n: r[...] for n, r in zip(_S1_CONST_NAMES, const_refs)}
    Bb = ef_ref.shape[0]
    M = D1
    BM = Bb * M
    P0 = (rows_ref[...] @ cr["isab0_AkvqT"]
          + wgt_ref[...] @ cr["Akvq_pe"] + cr["bkvq_pe"])
    X1 = _isab(P0, Bb, cr, "isab0")
    P1 = X1 @ cr["isab1_AkvqT"] + cr["isab1_bkvq"]
    X2 = _isab(P1, Bb, cr, "isab1")
    ef = ef_ref[...]
    Qpd = ef @ cr["WqdT"] + cr["bqd"]
    KVd = X2 @ cr["AkvdT"] + cr["bkvd"]
    Kpd = KVd[:, :DHID].reshape(Bb, M, DHID)
    Vpd = KVd[:, DHID:].reshape(Bb, M, DHID)
    Sd = ((Kpd * Qpd[:, None, :]).reshape(BM, DHID) @ cr["Mh"]).reshape(Bb, M, HP)
    Sd = Sd - jnp.max(Sd, axis=1, keepdims=True)
    Ad = jnp.exp(Sd)
    Ad = Ad / jnp.sum(Ad, axis=1, keepdims=True)
    Axd = (Ad.reshape(BM, HP) @ cr["Eh"]).reshape(Bb, M, DHID)
    Od = Qpd + jnp.sum(Axd * Vpd, axis=1)
    Od = Od + jnp.maximum(Od @ cr["WodT"] + cr["bod"], 0.0)
    efn = Od @ cr["WdlT"] + cr["bdl"]
    efn_ref[...] = efn
    kv_ref[...] = efn @ cr["WkvT"] + cr["bkv"]


def _stage1(rows, weight_f, efeat, consts, Bb):
    E = efeat.shape[0]
    grid = (E // Bb,)
    const_vals = [consts[n] for n in _S1_CONST_NAMES]

    def fullspec(a):
        return pl.BlockSpec(a.shape, lambda i: (0,) * a.ndim)

    return pl.pallas_call(
        _stage1_body,
        grid=grid,
        in_specs=[
            pl.BlockSpec((Bb * D1, IN_VDIM), lambda i: (i, 0)),
            pl.BlockSpec((Bb * D1, WDIM), lambda i: (i, 0)),
            pl.BlockSpec((Bb, IN_EDIM), lambda i: (i, 0)),
        ] + [fullspec(a) for a in const_vals],
        out_specs=[
            pl.BlockSpec((Bb, OUT_EDIM), lambda i: (i, 0)),
            pl.BlockSpec((Bb, KV_PAD), lambda i: (i, 0)),
        ],
        out_shape=[
            jax.ShapeDtypeStruct((E, OUT_EDIM), jnp.float32),
            jax.ShapeDtypeStruct((E, KV_PAD), jnp.float32),
        ],
        compiler_params=pltpu.CompilerParams(
            dimension_semantics=("arbitrary",)),
    )(rows, weight_f, efeat, *const_vals)


def _stage2_body(rows_ref, vf_ref, WqvT_ref, bqv_ref, out_ref):
    Bn = vf_ref.shape[0]
    M = D2
    q = vf_ref[...] @ WqvT_ref[...] + bqv_ref[...]
    rows = rows_ref[...]
    kn = rows[:, :DHID].reshape(Bn, M, DHID)
    s = jnp.sum(kn * q[:, None, :], axis=-1)
    s = jnp.where(s >= 0.0, s, 0.01 * s) * (1.0 / math.sqrt(DHID))
    s = s - jnp.max(s, axis=-1, keepdims=True)
    a = jnp.exp(s)
    a = a / jnp.sum(a, axis=-1, keepdims=True)
    vvn = rows[:, DHID:DHID + OUT_VDIM].reshape(Bn, M, OUT_VDIM)
    h = jnp.sum(a[:, :, None] * vvn, axis=1)
    out_ref[...] = jnp.maximum(h, 0.0)


def _stage2(rows2, vfeat, consts, Bn):
    N = vfeat.shape[0]
    grid = (N // Bn,)
    return pl.pallas_call(
        _stage2_body,
        grid=grid,
        in_specs=[
            pl.BlockSpec((Bn * D2, KV_PAD), lambda i: (i, 0)),
            pl.BlockSpec((Bn, IN_VDIM), lambda i: (i, 0)),
            pl.BlockSpec(consts["WqvT"].shape, lambda i: (0, 0)),
            pl.BlockSpec(consts["bqv"].shape, lambda i: (0, 0)),
        ],
        out_specs=pl.BlockSpec((Bn, OUT_VDIM), lambda i: (i, 0)),
        out_shape=jax.ShapeDtypeStruct((N, OUT_VDIM), jnp.float32),
        compiler_params=pltpu.CompilerParams(
            dimension_semantics=("arbitrary",)),
    )(rows2, vfeat, consts["WqvT"], consts["bqv"])


def kernel(vfeat, efeat, weight, params, nbr1, nbr2):
    consts = _prep_consts(params)
    idx1 = nbr1.astype(jnp.int32).reshape(-1)
    idx2 = nbr2.astype(jnp.int32).reshape(-1)
    rows1 = _sc_gather(vfeat, idx1, chunk=200)
    weight_f = weight.reshape(N_EDGES * D1, WDIM)
    efeat_new, kv = _stage1(rows1, weight_f, efeat, consts, Bb=200)
    rows2 = _sc_gather(kv, idx2, chunk=200)
    vfeat_new = _stage2(rows2, vfeat, consts, Bn=200)
    return (vfeat_new, efeat_new)
